# baseline (speedup 1.0000x reference)
import functools

import jax
import jax.numpy as jnp
from jax import lax
from jax.experimental import pallas as pl
from jax.experimental.pallas import tpu as pltpu
from jax.experimental.pallas import tpu_sc as plsc

B, L, C = 1024, 20, 128
NW = 32
BPW = B // NW


def _sc_pool_body(idx_hbm, table, out_hbm, idx_v, rows0, rows1, out_v,
                  sem0, sem1):
    epp = BPW // 2
    nc = plsc.get_sparse_core_info().num_cores
    wid = lax.axis_index("s") * nc + lax.axis_index("c")

    pltpu.sync_copy(idx_hbm.at[pl.ds(wid * BPW, BPW)], idx_v)

    rows = (rows0, rows1)
    sems = (sem0, sem1)

    def fire(h):
        buf, sem = rows[h % 2], sems[h % 2]
        return [
            pltpu.async_copy(table.at[wid].at[idx_v.at[h * epp + e]],
                             buf.at[pl.ds(e * L, L)], sem)
            for e in range(epp)
        ]

    pending = fire(0)
    for h in range(2):
        nxt = fire(h + 1) if h < 1 else []
        for cp in pending:
            cp.wait()
        pending = nxt

        buf = rows[h % 2]

        def body(e, carry, buf=buf, h=h):
            base = e * L
            for c in range(C // 16):
                sl = pl.ds(16 * c, 16)
                acc = buf[base, sl]
                for l in range(1, L):
                    acc = acc + buf[base + l, sl]
                out_v[h * epp + e, sl] = acc * (1.0 / L)
            return carry

        lax.fori_loop(0, epp, body, 0)

    pltpu.sync_copy(out_v, out_hbm.at[pl.ds(wid * BPW, BPW)])


def _sc_pool(style, sty_table_rep):
    mesh = plsc.VectorSubcoreMesh(core_axis_name="c", subcore_axis_name="s")
    return pl.kernel(
        _sc_pool_body,
        out_type=jax.ShapeDtypeStruct((B, C), jnp.float32),
        mesh=mesh,
        scratch_types=[
            pltpu.VMEM((BPW, L), jnp.int32),
            pltpu.VMEM((BPW * L // 2, C), jnp.float32),
            pltpu.VMEM((BPW * L // 2, C), jnp.float32),
            pltpu.VMEM((BPW, C), jnp.float32),
            pltpu.SemaphoreType.DMA,
            pltpu.SemaphoreType.DMA,
        ],
    )(style, sty_table_rep)


def _silu(x):
    return x * jax.nn.sigmoid(x)


def _mm(x, w):
    return jnp.dot(x, w, preferred_element_type=jnp.float32,
                   precision=lax.Precision.HIGHEST)


def _tc_dense_body(emb_ref, eps_ref, cat_ref, cat_t, gen_ref, gen_t,
                   cW1, cb1, cW2, cb2, gW1, gb1, gW2, gb2, sW1, sb1,
                   sW2, sb2, muW1, mub1, muW2, mub2, vW1, vb1, vW2, vb2,
                   out_ref):
    def pooled(idx_ref, table_ref, nv):
        blk_idx = idx_ref[...]
        iota = lax.broadcasted_iota(jnp.int32, (1, nv), 1)
        cnt = jnp.zeros((blk_idx.shape[0], nv), jnp.float32)
        for l in range(L):
            cnt = cnt + (blk_idx[:, l:l + 1] == iota).astype(jnp.float32)
        return _mm(cnt, table_ref[...]) * (1.0 / L)

    ecp = pooled(cat_ref, cat_t, 16)
    egp = pooled(gen_ref, gen_t, 64)
    ec = _silu(_mm(_silu(_mm(ecp, cW1[...]) + cb1[...]),
                   cW2[...]) + cb2[...])
    eg = _silu(_mm(_silu(_mm(egp, gW1[...]) + gb1[...]),
                   gW2[...]) + gb2[...])
    es = _silu(_mm(_silu(_mm(emb_ref[...], sW1[...]) + sb1[...]),
                   sW2[...]) + sb2[...])
    cat = jnp.concatenate([ec, eg, es], axis=1)
    mu = _mm(jax.nn.relu(_mm(cat, muW1[...]) + mub1[...]), muW2[...]) \
        + mub2[...]
    var = _mm(jax.nn.relu(_mm(cat, vW1[...]) + vb1[...]), vW2[...]) \
        + vb2[...]
    out_ref[...] = mu + jnp.exp(0.5 * var) * eps_ref[...]


def _tc_dense(emb, eps, category, cat_table, genre, gen_table, weights):
    full = lambda a: pl.BlockSpec(a.shape, lambda: (0,) * a.ndim)
    args = [emb, eps, category, cat_table, genre, gen_table] + weights
    return pl.pallas_call(
        _tc_dense_body,
        in_specs=[full(a) for a in args],
        out_specs=pl.BlockSpec((B, C), lambda: (0, 0)),
        out_shape=jax.ShapeDtypeStruct((B, C), jnp.float32),
    )(*args)


def kernel(category, genre, style, cat_table, gen_table, sty_table,
           cW1, cb1, cW2, cb2, gW1, gb1, gW2, gb2, sW1, sb1, sW2, sb2,
           muW1, mub1, muW2, mub2, vW1, vb1, vW2, vb2, eps):
    sty_rep = jnp.tile(sty_table[None], (NW, 1, 1))
    emb = _sc_pool(style, sty_rep)
    weights = [cW1, cb1.reshape(1, -1), cW2, cb2.reshape(1, -1),
               gW1, gb1.reshape(1, -1), gW2, gb2.reshape(1, -1),
               sW1, sb1.reshape(1, -1), sW2, sb2.reshape(1, -1),
               muW1, mub1.reshape(1, -1), muW2, mub2.reshape(1, -1),
               vW1, vb1.reshape(1, -1), vW2, vb2.reshape(1, -1)]
    return _tc_dense(emb, eps, category, cat_table, genre, gen_table,
                     weights)
